# baseline (speedup 1.0000x reference)
import jax
import jax.numpy as jnp
from jax import lax
from jax.experimental import pallas as pl
from jax.experimental.pallas import tpu as pltpu
from jax.experimental.pallas import tpu_sc as plsc

N = 1024
NTR = 819
NTE = 205
NTR_PAD = 832
LANES = 16
SHIFT = 12.0


def _main_kernel(seq1_ref, adj_ref, idxte_ref, counts_ref, Wstru_ref,
                 bstru_ref, Wgat_ref, attsrc_ref, attdst_ref, bgat_ref,
                 Wa1_ref, ba1_ref, Wa2_ref, ba2_ref, loss_ref, test_ref,
                 wgat_v, wa2_v, dma_sem):
    f32 = jnp.float32
    cg = pltpu.make_async_copy(Wgat_ref, wgat_v, dma_sem)
    cg.start()
    seq1 = seq1_ref[...]
    adj = adj_ref[...]

    h = jnp.maximum(
        lax.dot_general(seq1, Wstru_ref[...], (((1,), (1,)), ((), ())),
                        preferred_element_type=f32) + bstru_ref[...][None, :],
        0.0)
    cg.wait()
    ca = pltpu.make_async_copy(Wa2_ref, wa2_v, dma_sem)
    ca.start()
    xw = lax.dot_general(h, wgat_v[...].T, (((1,), (0,)), ((), ())),
                         preferred_element_type=f32)

    a_src = jnp.sum(xw * attsrc_ref[...][None, :], axis=1, keepdims=True)
    a_dst = lax.dot_general(attdst_ref[...][None, :], xw,
                            (((1,), (1,)), ((), ())),
                            preferred_element_type=f32)

    z = a_src + a_dst
    e = jnp.where(z >= 0.0, z, 0.2 * z)

    rows = lax.broadcasted_iota(jnp.int32, (N, N), 0)
    cols = lax.broadcasted_iota(jnp.int32, (N, N), 1)
    cnt = adj + jnp.where(rows == cols, 1.0, 0.0)

    w = cnt * jnp.exp(e - SHIFT)

    num = lax.dot_general(w, xw, (((0,), (0,)), ((), ())),
                          preferred_element_type=f32)
    ones = jnp.ones((N, 1), f32)
    den = lax.dot_general(w, ones, (((0,), (0,)), ((), ())),
                          preferred_element_type=f32)
    emb = num / den + bgat_ref[...][None, :]

    x = jnp.maximum(
        lax.dot_general(seq1, Wa1_ref[...], (((1,), (1,)), ((), ())),
                        preferred_element_type=f32) + ba1_ref[...][None, :],
        0.0)
    ca.wait()
    x_ = lax.dot_general(x, wa2_v[...].T, (((1,), (0,)), ((), ())),
                         preferred_element_type=f32) + ba2_ref[...][None, :]
    da = seq1 - x_
    attr_half = 0.5 * jnp.sqrt(jnp.sum(da * da, axis=1, keepdims=True))

    p = lax.dot_general(emb, emb, (((1,), (1,)), ((), ())),
                        preferred_element_type=f32)
    s = jax.nn.sigmoid(p)
    ds = adj - s
    stru_half = 0.5 * jnp.sqrt(jnp.sum(ds * ds, axis=1, keepdims=True))

    score = attr_half + stru_half

    counts = counts_ref[...][None, :]
    loss_ref[...] = lax.dot_general(counts, score, (((1,), (0,)), ((), ())),
                                    preferred_element_type=f32) / NTR

    idxte = idxte_ref[...][None, :]
    oh_te = (lax.broadcasted_iota(jnp.int32, (N, NTE), 0) == idxte)
    test_ref[...] = lax.dot_general(score, oh_te.astype(f32),
                                    (((0,), (0,)), ((), ())),
                                    preferred_element_type=f32)


def _count_body(idxtr_hbm, out_hbm, idx_v, tab_v, sem):
    is_worker = jnp.logical_and(lax.axis_index("c") == 0,
                                lax.axis_index("s") == 0)

    @pl.when(is_worker)
    def _():
        c1 = pltpu.make_async_copy(idxtr_hbm, idx_v.at[pl.ds(0, NTR)], sem)
        c1.start()

        def zero_body(i, carry):
            tab_v[pl.ds(i * LANES, LANES)] = jnp.zeros((LANES,), jnp.float32)
            return carry

        lax.fori_loop(0, N // LANES, zero_body, 0)
        c1.wait()

        lane = lax.iota(jnp.int32, LANES)
        one = jnp.ones((LANES,), jnp.float32)

        def scat_body(i, carry):
            valid = lane + i * LANES < NTR
            idx = jnp.where(valid, idx_v[pl.ds(i * LANES, LANES)], 0)
            plsc.addupdate_scatter(tab_v, [idx],
                                   jnp.where(valid, one, 0.0))
            return carry

        lax.fori_loop(0, NTR_PAD // LANES, scat_body, 0)
        pltpu.sync_copy(tab_v, out_hbm)


def _sc_counts(idxtr):
    return pl.kernel(
        _count_body,
        out_type=jax.ShapeDtypeStruct((N,), jnp.float32),
        mesh=plsc.VectorSubcoreMesh(core_axis_name="c",
                                    subcore_axis_name="s"),
        compiler_params=pltpu.CompilerParams(needs_layout_passes=False),
        scratch_types=[
            pltpu.VMEM((NTR_PAD,), jnp.int32),
            pltpu.VMEM((N,), jnp.float32),
            pltpu.SemaphoreType.DMA,
        ],
    )(idxtr)


def kernel(seq1, adj, idx_train, idx_test, W_stru, b_stru, W_gat, att_src,
           att_dst, b_gat, W_a1, b_a1, W_a2, b_a2):
    f32 = jnp.float32
    seq1 = jnp.asarray(seq1, f32).reshape(N, 128)
    adj = jnp.asarray(adj, f32).reshape(N, N)
    idxtr = jnp.asarray(idx_train, jnp.int32).reshape(NTR)
    idxte = jnp.asarray(idx_test, jnp.int32).reshape(NTE)

    counts = _sc_counts(idxtr)

    vm = pl.BlockSpec(memory_space=pltpu.MemorySpace.VMEM)
    anym = pl.BlockSpec(memory_space=pl.MemorySpace.ANY)
    loss2d, test2d = pl.pallas_call(
        _main_kernel,
        in_specs=[vm, vm, vm, vm, vm, vm, anym, vm, vm, vm, vm, vm, anym,
                  vm],
        out_shape=(
            jax.ShapeDtypeStruct((1, 1), f32),
            jax.ShapeDtypeStruct((1, NTE), f32),
        ),
        scratch_shapes=[
            pltpu.VMEM((128, 64), f32),
            pltpu.VMEM((128, 64), f32),
            pltpu.SemaphoreType.DMA,
        ],
    )(seq1, adj, idxte, counts, W_stru, b_stru, W_gat, att_src, att_dst,
      b_gat, W_a1, b_a1, W_a2, b_a2)

    return (loss2d.reshape(()), test2d.reshape(NTE))


# speedup vs baseline: 1.0427x; 1.0427x over previous
import jax
import jax.numpy as jnp
from jax import lax
from jax.experimental import pallas as pl
from jax.experimental.pallas import tpu as pltpu
from jax.experimental.pallas import tpu_sc as plsc

N = 1024
NTR = 819
NTE = 205
NTR_PAD = 832
LANES = 16
SHIFT = 12.0


def _main_kernel(seq1_ref, adj_ref, idxte_ref, counts_ref, Wstru_ref,
                 bstru_ref, Wgat_ref, attsrc_ref, attdst_ref, bgat_ref,
                 Wa1_ref, ba1_ref, Wa2_ref, ba2_ref, loss_ref, test_ref):
    f32 = jnp.float32
    seq1 = seq1_ref[...]
    adj = adj_ref[...]

    h = jnp.maximum(
        lax.dot_general(seq1, Wstru_ref[...], (((1,), (1,)), ((), ())),
                        preferred_element_type=f32) + bstru_ref[...][None, :],
        0.0)
    xw = lax.dot_general(h, Wgat_ref[...].T, (((1,), (0,)), ((), ())),
                         preferred_element_type=f32)

    a_src = jnp.sum(xw * attsrc_ref[...][None, :], axis=1, keepdims=True)
    a_dst = lax.dot_general(attdst_ref[...][None, :], xw,
                            (((1,), (1,)), ((), ())),
                            preferred_element_type=f32)

    z = a_src + a_dst
    e = jnp.where(z >= 0.0, z, 0.2 * z)

    rows = lax.broadcasted_iota(jnp.int32, (N, N), 0)
    cols = lax.broadcasted_iota(jnp.int32, (N, N), 1)
    cnt = adj + jnp.where(rows == cols, 1.0, 0.0)

    w = cnt * jnp.exp(e - SHIFT)

    num = lax.dot_general(w, xw, (((0,), (0,)), ((), ())),
                          preferred_element_type=f32)
    ones = jnp.ones((N, 1), f32)
    den = lax.dot_general(w, ones, (((0,), (0,)), ((), ())),
                          preferred_element_type=f32)
    emb = num / den + bgat_ref[...][None, :]

    x = jnp.maximum(
        lax.dot_general(seq1, Wa1_ref[...], (((1,), (1,)), ((), ())),
                        preferred_element_type=f32) + ba1_ref[...][None, :],
        0.0)
    x_ = lax.dot_general(x, Wa2_ref[...].T, (((1,), (0,)), ((), ())),
                         preferred_element_type=f32) + ba2_ref[...][None, :]
    da = seq1 - x_
    attr_half = 0.5 * jnp.sqrt(jnp.sum(da * da, axis=1, keepdims=True))

    p = lax.dot_general(emb, emb, (((1,), (1,)), ((), ())),
                        preferred_element_type=f32)
    s = jax.nn.sigmoid(p)
    ds = adj - s
    stru_half = 0.5 * jnp.sqrt(jnp.sum(ds * ds, axis=1, keepdims=True))

    score = attr_half + stru_half

    counts = counts_ref[...][None, :]
    loss_ref[...] = lax.dot_general(counts, score, (((1,), (0,)), ((), ())),
                                    preferred_element_type=f32) / NTR

    idxte = idxte_ref[...][None, :]
    oh_te = (lax.broadcasted_iota(jnp.int32, (N, NTE), 0) == idxte)
    test_ref[...] = lax.dot_general(score, oh_te.astype(f32),
                                    (((0,), (0,)), ((), ())),
                                    preferred_element_type=f32)


def _count_body(idxtr_hbm, out_hbm, idx_v, tab_v, sem):
    is_worker = jnp.logical_and(lax.axis_index("c") == 0,
                                lax.axis_index("s") == 0)

    @pl.when(is_worker)
    def _():
        c1 = pltpu.make_async_copy(idxtr_hbm, idx_v.at[pl.ds(0, NTR)], sem)
        c1.start()

        def zero_body(i, carry):
            tab_v[pl.ds(i * LANES, LANES)] = jnp.zeros((LANES,), jnp.float32)
            return carry

        lax.fori_loop(0, N // LANES, zero_body, 0)
        c1.wait()

        lane = lax.iota(jnp.int32, LANES)
        one = jnp.ones((LANES,), jnp.float32)

        def scat_body(i, carry):
            valid = lane + i * LANES < NTR
            idx = jnp.where(valid, idx_v[pl.ds(i * LANES, LANES)], 0)
            plsc.addupdate_scatter(tab_v, [idx],
                                   jnp.where(valid, one, 0.0))
            return carry

        lax.fori_loop(0, NTR_PAD // LANES, scat_body, 0)
        pltpu.sync_copy(tab_v, out_hbm)


def _sc_counts(idxtr):
    return pl.kernel(
        _count_body,
        out_type=jax.ShapeDtypeStruct((N,), jnp.float32),
        mesh=plsc.VectorSubcoreMesh(core_axis_name="c",
                                    subcore_axis_name="s"),
        compiler_params=pltpu.CompilerParams(needs_layout_passes=False),
        scratch_types=[
            pltpu.VMEM((NTR_PAD,), jnp.int32),
            pltpu.VMEM((N,), jnp.float32),
            pltpu.SemaphoreType.DMA,
        ],
    )(idxtr)


def kernel(seq1, adj, idx_train, idx_test, W_stru, b_stru, W_gat, att_src,
           att_dst, b_gat, W_a1, b_a1, W_a2, b_a2):
    f32 = jnp.float32
    seq1 = jnp.asarray(seq1, f32).reshape(N, 128)
    adj = jnp.asarray(adj, f32).reshape(N, N)
    idxtr = jnp.asarray(idx_train, jnp.int32).reshape(NTR)
    idxte = jnp.asarray(idx_test, jnp.int32).reshape(NTE)

    counts = _sc_counts(idxtr)

    loss2d, test2d = pl.pallas_call(
        _main_kernel,
        out_shape=(
            jax.ShapeDtypeStruct((1, 1), f32),
            jax.ShapeDtypeStruct((1, NTE), f32),
        ),
    )(seq1, adj, idxte, counts, W_stru, b_stru, W_gat, att_src, att_dst,
      b_gat, W_a1, b_a1, W_a2, b_a2)

    return (loss2d.reshape(()), test2d.reshape(NTE))


# speedup vs baseline: 1.0442x; 1.0014x over previous
import jax
import jax.numpy as jnp
from jax import lax
from jax.experimental import pallas as pl
from jax.experimental.pallas import tpu as pltpu
from jax.experimental.pallas import tpu_sc as plsc

N = 1024
NTR = 819
NTE = 205
NTR_PAD = 832
LANES = 16
SHIFT = 12.0


def _main_kernel(seq1_ref, adj_ref, idxte_ref, counts_ref, Wstru_ref,
                 bstru_ref, Wgat_ref, attsrc_ref, attdst_ref, bgat_ref,
                 Wa1_ref, ba1_ref, Wa2_ref, ba2_ref, loss_ref, test_ref):
    f32 = jnp.float32
    seq1 = seq1_ref[...]
    adj = adj_ref[...]

    h = jnp.maximum(
        lax.dot_general(seq1, Wstru_ref[...], (((1,), (1,)), ((), ())),
                        preferred_element_type=f32) + bstru_ref[...][None, :],
        0.0)
    xw = lax.dot_general(h, Wgat_ref[...].T, (((1,), (0,)), ((), ())),
                         preferred_element_type=f32)

    a_src = jnp.sum(xw * attsrc_ref[...][None, :], axis=1, keepdims=True)
    a_dst = lax.dot_general(attdst_ref[...][None, :], xw,
                            (((1,), (1,)), ((), ())),
                            preferred_element_type=f32)

    z = a_src + a_dst
    e = jnp.where(z >= 0.0, z, 0.2 * z)

    rows = lax.broadcasted_iota(jnp.int32, (N, N), 0)
    cols = lax.broadcasted_iota(jnp.int32, (N, N), 1)
    cnt = adj + jnp.where(rows == cols, 1.0, 0.0)

    w = (cnt * jnp.exp(e - SHIFT)).astype(jnp.bfloat16)

    xw16 = xw.astype(jnp.bfloat16)
    num = lax.dot_general(w, xw16, (((0,), (0,)), ((), ())),
                          preferred_element_type=f32)
    ones = jnp.ones((N, 1), jnp.bfloat16)
    den = lax.dot_general(w, ones, (((0,), (0,)), ((), ())),
                          preferred_element_type=f32)
    emb = num / den + bgat_ref[...][None, :]

    x = jnp.maximum(
        lax.dot_general(seq1, Wa1_ref[...], (((1,), (1,)), ((), ())),
                        preferred_element_type=f32) + ba1_ref[...][None, :],
        0.0)
    x_ = lax.dot_general(x, Wa2_ref[...].T, (((1,), (0,)), ((), ())),
                         preferred_element_type=f32) + ba2_ref[...][None, :]
    da = seq1 - x_
    attr_half = 0.5 * jnp.sqrt(jnp.sum(da * da, axis=1, keepdims=True))

    emb16 = emb.astype(jnp.bfloat16)
    p = lax.dot_general(emb16, emb16, (((1,), (1,)), ((), ())),
                        preferred_element_type=f32)
    s = jax.nn.sigmoid(p)
    ds = adj - s
    stru_half = 0.5 * jnp.sqrt(jnp.sum(ds * ds, axis=1, keepdims=True))

    score = attr_half + stru_half

    counts = counts_ref[...][None, :]
    loss_ref[...] = lax.dot_general(counts, score, (((1,), (0,)), ((), ())),
                                    preferred_element_type=f32) / NTR

    idxte = idxte_ref[...][None, :]
    oh_te = (lax.broadcasted_iota(jnp.int32, (N, NTE), 0) == idxte)
    test_ref[...] = lax.dot_general(score, oh_te.astype(f32),
                                    (((0,), (0,)), ((), ())),
                                    preferred_element_type=f32)


def _count_body(idxtr_hbm, out_hbm, idx_v, tab_v, sem):
    is_worker = jnp.logical_and(lax.axis_index("c") == 0,
                                lax.axis_index("s") == 0)

    @pl.when(is_worker)
    def _():
        c1 = pltpu.make_async_copy(idxtr_hbm, idx_v.at[pl.ds(0, NTR)], sem)
        c1.start()

        def zero_body(i, carry):
            tab_v[pl.ds(i * LANES, LANES)] = jnp.zeros((LANES,), jnp.float32)
            return carry

        lax.fori_loop(0, N // LANES, zero_body, 0)
        c1.wait()

        lane = lax.iota(jnp.int32, LANES)
        one = jnp.ones((LANES,), jnp.float32)

        def scat_body(i, carry):
            valid = lane + i * LANES < NTR
            idx = jnp.where(valid, idx_v[pl.ds(i * LANES, LANES)], 0)
            plsc.addupdate_scatter(tab_v, [idx],
                                   jnp.where(valid, one, 0.0))
            return carry

        lax.fori_loop(0, NTR_PAD // LANES, scat_body, 0)
        pltpu.sync_copy(tab_v, out_hbm)


def _sc_counts(idxtr):
    return pl.kernel(
        _count_body,
        out_type=jax.ShapeDtypeStruct((N,), jnp.float32),
        mesh=plsc.VectorSubcoreMesh(core_axis_name="c",
                                    subcore_axis_name="s"),
        compiler_params=pltpu.CompilerParams(needs_layout_passes=False),
        scratch_types=[
            pltpu.VMEM((NTR_PAD,), jnp.int32),
            pltpu.VMEM((N,), jnp.float32),
            pltpu.SemaphoreType.DMA,
        ],
    )(idxtr)


def kernel(seq1, adj, idx_train, idx_test, W_stru, b_stru, W_gat, att_src,
           att_dst, b_gat, W_a1, b_a1, W_a2, b_a2):
    f32 = jnp.float32
    seq1 = jnp.asarray(seq1, f32).reshape(N, 128)
    adj = jnp.asarray(adj, f32).reshape(N, N)
    idxtr = jnp.asarray(idx_train, jnp.int32).reshape(NTR)
    idxte = jnp.asarray(idx_test, jnp.int32).reshape(NTE)

    counts = _sc_counts(idxtr)

    loss2d, test2d = pl.pallas_call(
        _main_kernel,
        out_shape=(
            jax.ShapeDtypeStruct((1, 1), f32),
            jax.ShapeDtypeStruct((1, NTE), f32),
        ),
    )(seq1, adj, idxte, counts, W_stru, b_stru, W_gat, att_src, att_dst,
      b_gat, W_a1, b_a1, W_a2, b_a2)

    return (loss2d.reshape(()), test2d.reshape(NTE))


# speedup vs baseline: 1.0555x; 1.0108x over previous
import jax
import jax.numpy as jnp
from jax import lax
from jax.experimental import pallas as pl
from jax.experimental.pallas import tpu as pltpu
from jax.experimental.pallas import tpu_sc as plsc

N = 1024
NTR = 819
NTE = 205
NTR_PAD = 832
LANES = 16
SHIFT = 12.0


def _main_kernel(seq1_ref, adj_ref, idxte_ref, counts_ref, Wstru_ref,
                 bstru_ref, Wgat_ref, attsrc_ref, attdst_ref, bgat_ref,
                 Wa1_ref, ba1_ref, Wa2_ref, ba2_ref, loss_ref, test_ref):
    f32 = jnp.float32
    seq1 = seq1_ref[...]
    adj = adj_ref[...]

    h = jnp.maximum(
        lax.dot_general(seq1, Wstru_ref[...], (((1,), (1,)), ((), ())),
                        preferred_element_type=f32) + bstru_ref[...][None, :],
        0.0)
    xw = lax.dot_general(h, Wgat_ref[...].T, (((1,), (0,)), ((), ())),
                         preferred_element_type=f32)

    a_src = jnp.sum(xw * attsrc_ref[...][None, :], axis=1, keepdims=True)
    a_dst = lax.dot_general(attdst_ref[...][None, :], xw,
                            (((1,), (1,)), ((), ())),
                            preferred_element_type=f32)

    z = a_src + a_dst
    e = jnp.where(z >= 0.0, z, 0.2 * z)

    rows = lax.broadcasted_iota(jnp.int32, (N, N), 0)
    cols = lax.broadcasted_iota(jnp.int32, (N, N), 1)
    cnt = adj + jnp.where(rows == cols, 1.0, 0.0)

    w = cnt * jnp.exp(e - SHIFT)

    num = lax.dot_general(w, xw, (((0,), (0,)), ((), ())),
                          preferred_element_type=f32)
    ones = jnp.ones((N, 1), f32)
    den = lax.dot_general(w, ones, (((0,), (0,)), ((), ())),
                          preferred_element_type=f32)
    emb = num / den + bgat_ref[...][None, :]

    x = jnp.maximum(
        lax.dot_general(seq1, Wa1_ref[...], (((1,), (1,)), ((), ())),
                        preferred_element_type=f32) + ba1_ref[...][None, :],
        0.0)
    x_ = lax.dot_general(x, Wa2_ref[...].T, (((1,), (0,)), ((), ())),
                         preferred_element_type=f32) + ba2_ref[...][None, :]
    da = seq1 - x_
    attr_half = 0.5 * jnp.sqrt(jnp.sum(da * da, axis=1, keepdims=True))

    p = lax.dot_general(emb, emb, (((1,), (1,)), ((), ())),
                        preferred_element_type=f32)
    s = jax.nn.sigmoid(p)
    ds = adj - s
    stru_half = 0.5 * jnp.sqrt(jnp.sum(ds * ds, axis=1, keepdims=True))

    score = attr_half + stru_half

    counts = counts_ref[...][None, :]
    loss_ref[...] = lax.dot_general(counts, score, (((1,), (0,)), ((), ())),
                                    preferred_element_type=f32) / NTR

    idxte = idxte_ref[...][None, :]
    oh_te = (lax.broadcasted_iota(jnp.int32, (N, NTE), 0) == idxte)
    test_ref[...] = lax.dot_general(score, oh_te.astype(f32),
                                    (((0,), (0,)), ((), ())),
                                    preferred_element_type=f32)


def _count_body(idxtr_hbm, out_hbm, idx_v, tab_v, sem):
    is_worker = jnp.logical_and(lax.axis_index("c") == 1,
                                lax.axis_index("s") == 0)

    @pl.when(is_worker)
    def _():
        c1 = pltpu.make_async_copy(idxtr_hbm, idx_v.at[pl.ds(0, NTR)], sem)
        c1.start()

        def zero_body(i, carry):
            tab_v[pl.ds(i * LANES, LANES)] = jnp.zeros((LANES,), jnp.float32)
            return carry

        lax.fori_loop(0, N // LANES, zero_body, 0)
        c1.wait()

        lane = lax.iota(jnp.int32, LANES)
        one = jnp.ones((LANES,), jnp.float32)

        def scat_body(i, carry):
            valid = lane + i * LANES < NTR
            idx = jnp.where(valid, idx_v[pl.ds(i * LANES, LANES)], 0)
            plsc.addupdate_scatter(tab_v, [idx],
                                   jnp.where(valid, one, 0.0))
            return carry

        lax.fori_loop(0, NTR_PAD // LANES, scat_body, 0)
        pltpu.sync_copy(tab_v, out_hbm)


def _sc_counts(idxtr):
    return pl.kernel(
        _count_body,
        out_type=jax.ShapeDtypeStruct((N,), jnp.float32),
        mesh=plsc.VectorSubcoreMesh(core_axis_name="c",
                                    subcore_axis_name="s"),
        compiler_params=pltpu.CompilerParams(needs_layout_passes=False),
        scratch_types=[
            pltpu.VMEM((NTR_PAD,), jnp.int32),
            pltpu.VMEM((N,), jnp.float32),
            pltpu.SemaphoreType.DMA,
        ],
    )(idxtr)


def kernel(seq1, adj, idx_train, idx_test, W_stru, b_stru, W_gat, att_src,
           att_dst, b_gat, W_a1, b_a1, W_a2, b_a2):
    f32 = jnp.float32
    seq1 = jnp.asarray(seq1, f32).reshape(N, 128)
    adj = jnp.asarray(adj, f32).reshape(N, N)
    idxtr = jnp.asarray(idx_train, jnp.int32).reshape(NTR)
    idxte = jnp.asarray(idx_test, jnp.int32).reshape(NTE)

    counts = _sc_counts(idxtr)

    loss2d, test2d = pl.pallas_call(
        _main_kernel,
        out_shape=(
            jax.ShapeDtypeStruct((1, 1), f32),
            jax.ShapeDtypeStruct((1, NTE), f32),
        ),
    )(seq1, adj, idxte, counts, W_stru, b_stru, W_gat, att_src, att_dst,
      b_gat, W_a1, b_a1, W_a2, b_a2)

    return (loss2d.reshape(()), test2d.reshape(NTE))
